# baseline (speedup 1.0000x reference)
import functools

import jax
import jax.numpy as jnp
from jax import lax
from jax.experimental import pallas as pl
from jax.experimental.pallas import tpu as pltpu
from jax.experimental.pallas import tpu_sc as plsc

VOCAB = 1000000
EMBED = 64
B = 4096
L = 200

NC = 2
NS = 16
NW = NC * NS
ROWS_PER_W = B // NW
HALF = L // 2


def _worker_body(tok_hbm, wE_hbm, pos_hbm, out_hbm,
                 idx_v, pos_v, g0, g1, o0, o1, gs0, gs1, ws0, ws1):
    cid = lax.axis_index("c")
    sid = lax.axis_index("s")
    wid = sid * NC + cid
    base = wid * ROWS_PER_W
    ibase = wid * (2 * ROWS_PER_W)

    pltpu.sync_copy(tok_hbm.at[pl.ds(ibase, 2 * ROWS_PER_W)], idx_v)
    pltpu.sync_copy(pos_hbm, pos_v)

    gbuf = (g0, g1)
    obuf = (o0, o1)
    gsem = (gs0, gs1)
    wsem = (ws0, ws1)

    def gather_copies(cc, b):
        return (
            pltpu.make_async_copy(
                wE_hbm.at[idx_v.at[2 * cc]], gbuf[b].at[pl.ds(0, HALF)],
                gsem[b]),
            pltpu.make_async_copy(
                wE_hbm.at[idx_v.at[2 * cc + 1]], gbuf[b].at[pl.ds(HALF, HALF)],
                gsem[b]),
        )

    def write_copy(cc, b):
        return pltpu.make_async_copy(obuf[b], out_hbm.at[base + cc], wsem[b])

    def start_gather(cc, b):
        for c in gather_copies(cc, b):
            c.start()

    def chunk_step(cc, b):
        for c in gather_copies(cc, b):
            c.wait()

        @pl.when(cc >= 2)
        def _():
            write_copy(cc - 2, b).wait()

        def add_row(j, carry):
            for k in range(EMBED // 16):
                sl = pl.ds(16 * k, 16)
                obuf[b][j, sl] = gbuf[b][j, sl] + pos_v[j, sl]
            return carry

        lax.fori_loop(0, L, add_row, 0)

        @pl.when(cc + 2 < ROWS_PER_W)
        def _():
            start_gather(cc + 2, b)

        write_copy(cc, b).start()

    start_gather(0, 0)
    start_gather(1, 1)

    def loop_body(i, carry):
        chunk_step(2 * i, 0)
        chunk_step(2 * i + 1, 1)
        return carry

    lax.fori_loop(0, ROWS_PER_W // 2, loop_body, 0)

    write_copy(ROWS_PER_W - 2, 0).wait()
    write_copy(ROWS_PER_W - 1, 1).wait()


def _sc_embed(tok, W_E, W_pos):
    mesh = plsc.VectorSubcoreMesh(core_axis_name="c", subcore_axis_name="s")
    kern = functools.partial(
        pl.kernel,
        out_type=jax.ShapeDtypeStruct((B, L, EMBED), jnp.float32),
        mesh=mesh,
        scratch_types=[
            pltpu.VMEM((2 * ROWS_PER_W, HALF), jnp.int32),
            pltpu.VMEM((L, EMBED), jnp.float32),
            pltpu.VMEM((L, EMBED), jnp.float32),
            pltpu.VMEM((L, EMBED), jnp.float32),
            pltpu.VMEM((L, EMBED), jnp.float32),
            pltpu.VMEM((L, EMBED), jnp.float32),
            pltpu.SemaphoreType.DMA,
            pltpu.SemaphoreType.DMA,
            pltpu.SemaphoreType.DMA,
            pltpu.SemaphoreType.DMA,
        ],
        compiler_params=pltpu.CompilerParams(use_tc_tiling_on_sc=False),
    )(_worker_body)
    return kern(tok, W_E, W_pos)


def kernel(tokens, W_E, W_pos):
    tok = tokens.reshape(B * 2, HALF).astype(jnp.int32)
    return _sc_embed(tok, W_E, W_pos)
